# baseline (speedup 1.0000x reference)
import functools

import jax
import jax.numpy as jnp
from jax import lax
from jax.experimental import pallas as pl
from jax.experimental.pallas import tpu as pltpu
from jax.experimental.pallas import tpu_sc as plsc

N_USERS = 1000000
N_ITEMS = 1000000
EMBED_DIM = 32
BATCH = 16384

_info = plsc.get_sparse_core_info()
NC = _info.num_cores
NS = _info.num_subcores
L = _info.num_lanes
NW = NC * NS
B_PER_W = BATCH // NW
IDX_CHUNK = 128
N_CHUNKS = B_PER_W // IDX_CHUNK


def _gau_body(uids_hbm, iids_hbm, ut_hbm, it_hbm, ubt_hbm, ibt_hbm,
              out_hbm, uidx_v, iidx_v, urows_v, irows_v, ub_v, ib_v,
              out_v, tr_v, sem):
    wid = lax.axis_index("s") * NC + lax.axis_index("c")
    base = wid * B_PER_W

    pltpu.sync_copy(uids_hbm.at[wid], uidx_v)
    pltpu.sync_copy(iids_hbm.at[wid], iidx_v)

    copies = []
    for j in range(N_CHUNKS):
        sl = pl.ds(j * IDX_CHUNK, IDX_CHUNK)
        copies.append(pltpu.async_copy(
            ut_hbm.at[uidx_v.at[j]], urows_v.at[sl], sem))
        copies.append(pltpu.async_copy(
            it_hbm.at[iidx_v.at[j]], irows_v.at[sl], sem))
        copies.append(pltpu.async_copy(
            ubt_hbm.at[uidx_v.at[j]], ub_v.at[sl], sem))
        copies.append(pltpu.async_copy(
            ibt_hbm.at[iidx_v.at[j]], ib_v.at[sl], sem))
    for c in copies:
        c.wait()

    perm = lax.iota(jnp.int32, L) * L

    def group(g, _):
        rbase = g * L
        for r in range(L):
            b = rbase + r
            u0 = urows_v[b, pl.ds(0, L)]
            u1 = urows_v[b, pl.ds(L, L)]
            i0 = irows_v[b, pl.ds(0, L)]
            i1 = irows_v[b, pl.ds(L, L)]
            p = u0 * i0 + u1 * i1
            plsc.store_scatter(tr_v, [perm + r], p)
        acc = ub_v[pl.ds(rbase, L)] + ib_v[pl.ds(rbase, L)]
        for j in range(L):
            acc = acc + tr_v[pl.ds(j * L, L)]
        out_v[pl.ds(rbase, L)] = acc
        return 0

    lax.fori_loop(0, B_PER_W // L, group, 0)

    pltpu.sync_copy(out_v, out_hbm.at[pl.ds(base, B_PER_W)])


@jax.jit
def _gau_sc(uids_r, iids_r, user_table, item_table, ub_1d, ib_1d):
    mesh = plsc.VectorSubcoreMesh(core_axis_name="c", subcore_axis_name="s")
    k = functools.partial(
        pl.kernel,
        mesh=mesh,
        compiler_params=pltpu.CompilerParams(
            needs_layout_passes=False, use_tc_tiling_on_sc=False),
        out_type=jax.ShapeDtypeStruct((BATCH,), jnp.float32),
        scratch_types=[
            pltpu.VMEM((N_CHUNKS, IDX_CHUNK), jnp.int32),
            pltpu.VMEM((N_CHUNKS, IDX_CHUNK), jnp.int32),
            pltpu.VMEM((B_PER_W, EMBED_DIM), jnp.float32),
            pltpu.VMEM((B_PER_W, EMBED_DIM), jnp.float32),
            pltpu.VMEM((B_PER_W,), jnp.float32),
            pltpu.VMEM((B_PER_W,), jnp.float32),
            pltpu.VMEM((B_PER_W,), jnp.float32),
            pltpu.VMEM((L * L,), jnp.float32),
            pltpu.SemaphoreType.DMA,
        ],
    )(_gau_body)
    return k(uids_r, iids_r, user_table, item_table, ub_1d, ib_1d)


def kernel(uids, iids, user_table, item_table, user_bias_table, item_bias_table):
    uids_r = uids.astype(jnp.int32).reshape(NW, N_CHUNKS, IDX_CHUNK)
    iids_r = iids.astype(jnp.int32).reshape(NW, N_CHUNKS, IDX_CHUNK)
    ub_1d = user_bias_table.reshape(N_USERS)
    ib_1d = item_bias_table.reshape(N_ITEMS)
    return _gau_sc(uids_r, iids_r, user_table, item_table, ub_1d, ib_1d)
